# Initial kernel scaffold; baseline (speedup 1.0000x reference)
import jax
import jax.numpy as jnp
from jax.experimental import pallas as pl


def kernel(x, We, be, Wg, bg):
    raise NotImplementedError("write your pallas kernel here")


# speedup vs baseline: 2.5015x; 2.5015x over previous
import functools

import jax
import jax.numpy as jnp
from jax.experimental import pallas as pl

_NUM_EXPERTS = 8
_IN = 768
_OUT = 768
_BN = 512


def _moe_block_kernel(x_ref, wt_ref, be_ref, wgt_ref, bg_ref, out_ref):
    x = x_ref[...]
    logits = jnp.dot(x, wgt_ref[...], preferred_element_type=jnp.float32)
    logits = logits + bg_ref[...]
    m = jnp.max(logits, axis=1, keepdims=True)
    ex = jnp.exp(logits - m)
    g = ex / jnp.sum(ex, axis=1, keepdims=True)

    y = jnp.dot(x, wt_ref[...], preferred_element_type=jnp.float32)

    acc = jnp.dot(g, be_ref[...], preferred_element_type=jnp.float32)
    for e in range(_NUM_EXPERTS):
        acc = acc + g[:, e:e + 1] * y[:, e * _OUT:(e + 1) * _OUT]
    out_ref[...] = acc


@jax.jit
def kernel(x, We, be, Wg, bg):
    n = x.shape[0]
    wt = We.transpose(2, 0, 1).reshape(_IN, _NUM_EXPERTS * _OUT)
    wgt = Wg.T
    bg2 = bg.reshape(1, _NUM_EXPERTS)

    grid = (n // _BN,)
    return pl.pallas_call(
        _moe_block_kernel,
        grid=grid,
        in_specs=[
            pl.BlockSpec((_BN, _IN), lambda i: (i, 0)),
            pl.BlockSpec((_IN, _NUM_EXPERTS * _OUT), lambda i: (0, 0)),
            pl.BlockSpec((_NUM_EXPERTS, _OUT), lambda i: (0, 0)),
            pl.BlockSpec((_IN, _NUM_EXPERTS), lambda i: (0, 0)),
            pl.BlockSpec((1, _NUM_EXPERTS), lambda i: (0, 0)),
        ],
        out_specs=pl.BlockSpec((_BN, _OUT), lambda i: (i, 0)),
        out_shape=jax.ShapeDtypeStruct((n, _OUT), jnp.float32),
    )(x, wt, be, wgt, bg2)


# speedup vs baseline: 2.5200x; 1.0074x over previous
import functools

import jax
import jax.numpy as jnp
from jax.experimental import pallas as pl

_NUM_EXPERTS = 8
_IN = 768
_OUT = 768
_BN = 512


def _moe_block_kernel(x_ref, wt_ref, be_ref, wgt_ref, bg_ref, out_ref):
    x = x_ref[...]
    xb = x.astype(jnp.bfloat16)
    logits = jnp.dot(x, wgt_ref[...], preferred_element_type=jnp.float32)
    logits = logits + bg_ref[...]
    m = jnp.max(logits, axis=1, keepdims=True)
    ex = jnp.exp(logits - m)
    g = ex / jnp.sum(ex, axis=1, keepdims=True)

    y = jnp.dot(xb, wt_ref[...].astype(jnp.bfloat16),
                preferred_element_type=jnp.float32)

    acc = jnp.dot(g, be_ref[...], preferred_element_type=jnp.float32)
    for e in range(_NUM_EXPERTS):
        acc = acc + g[:, e:e + 1] * y[:, e * _OUT:(e + 1) * _OUT]
    out_ref[...] = acc


@jax.jit
def kernel(x, We, be, Wg, bg):
    n = x.shape[0]
    wt = We.transpose(2, 0, 1).reshape(_IN, _NUM_EXPERTS * _OUT)
    wgt = Wg.T
    bg2 = bg.reshape(1, _NUM_EXPERTS)

    grid = (n // _BN,)
    return pl.pallas_call(
        _moe_block_kernel,
        grid=grid,
        in_specs=[
            pl.BlockSpec((_BN, _IN), lambda i: (i, 0)),
            pl.BlockSpec((_IN, _NUM_EXPERTS * _OUT), lambda i: (0, 0)),
            pl.BlockSpec((_NUM_EXPERTS, _OUT), lambda i: (0, 0)),
            pl.BlockSpec((_IN, _NUM_EXPERTS), lambda i: (0, 0)),
            pl.BlockSpec((1, _NUM_EXPERTS), lambda i: (0, 0)),
        ],
        out_specs=pl.BlockSpec((_BN, _OUT), lambda i: (i, 0)),
        out_shape=jax.ShapeDtypeStruct((n, _OUT), jnp.float32),
    )(x, wt, be, wgt, bg2)
